# baseline (speedup 1.0000x reference)
import functools

import jax
import jax.numpy as jnp
from jax import lax
from jax.experimental import pallas as pl
from jax.experimental.pallas import tpu as pltpu
from jax.experimental.pallas import tpu_sc as plsc

_N = 4096
_NCOMBO = 4
_SC_Q = 1024
_CHUNK = _SC_Q // 8
_GBLK = 8
_TC_TILE = 256


def _newton_sqrt(x):
    i = lax.bitcast_convert_type(x, jnp.int32)
    y = lax.bitcast_convert_type(
        (i >> 1) + jnp.int32(0x1FBD3F7D), jnp.float32)
    half = jnp.float32(0.5)
    y = half * (y + x / y)
    y = half * (y + x / y)
    y = half * (y + x / y)
    return y


def _chamfer_sc(q_hbm, out_hbm, qx, qy, qz, cx, cy, cz, accv):
    nc = 2
    wid = lax.axis_index("s") * nc + lax.axis_index("c")
    combo = wid // 8
    chunk = wid % 8
    ccombo = combo ^ 1
    qbase = chunk * _CHUNK

    qrow = combo * (3 * _N)
    crow = ccombo * (3 * _N)
    pltpu.sync_copy(q_hbm.at[pl.ds(qrow + qbase, _CHUNK)], qx)
    pltpu.sync_copy(q_hbm.at[pl.ds(qrow + _N + qbase, _CHUNK)], qy)
    pltpu.sync_copy(q_hbm.at[pl.ds(qrow + 2 * _N + qbase, _CHUNK)], qz)
    pltpu.sync_copy(q_hbm.at[pl.ds(crow, _N)], cx)
    pltpu.sync_copy(q_hbm.at[pl.ds(crow + _N, _N)], cy)
    pltpu.sync_copy(q_hbm.at[pl.ds(crow + 2 * _N, _N)], cz)

    acc = jnp.zeros((16,), jnp.float32)
    big = jnp.full((16,), 3.0e38, jnp.float32)

    for blk in range(_CHUNK // 16 // _GBLK):
        qvs = []
        for g in range(_GBLK):
            off = (blk * _GBLK + g) * 16
            qvs.append((qx[pl.ds(off, 16)],
                        qy[pl.ds(off, 16)],
                        qz[pl.ds(off, 16)]))

        def body(jb, dmins, qvs=qvs):
            base = jb * 16
            cxv = cx[pl.ds(base, 16)]
            cyv = cy[pl.ds(base, 16)]
            czv = cz[pl.ds(base, 16)]
            out = list(dmins)
            for lane in range(16):
                bx = cxv[lane]
                by = cyv[lane]
                bz = czv[lane]
                for g in range(_GBLK):
                    dx = qvs[g][0] - bx
                    dy = qvs[g][1] - by
                    dz = qvs[g][2] - bz
                    d2 = dx * dx + dy * dy + dz * dz
                    out[g] = jnp.minimum(out[g], d2)
            return tuple(out)

        dmins = lax.fori_loop(0, _N // 16, body, tuple([big] * _GBLK))
        for g in range(_GBLK):
            acc = acc + _newton_sqrt(dmins[g])

    accv[...] = acc
    pltpu.sync_copy(accv, out_hbm.at[pl.ds(wid * 16, 16)])


def _chamfer_tc(q_ref, c_ref, o_ref):
    q = q_ref[0]
    c = c_ref[0]
    t = q.shape[0]
    qs = q * (-2.0)
    qs_hi = qs.astype(jnp.bfloat16)
    qs_lo = (qs - qs_hi.astype(jnp.float32)).astype(jnp.bfloat16)
    c_hi = c.astype(jnp.bfloat16)
    c_lo = (c - c_hi.astype(jnp.float32)).astype(jnp.bfloat16)
    c2 = jnp.sum(c * c, axis=1, keepdims=True)
    c2_hi = c2.astype(jnp.bfloat16)
    c2_lo = (c2 - c2_hi.astype(jnp.float32)).astype(jnp.bfloat16)
    ones = jnp.ones((t, 1), jnp.bfloat16)
    qcat = jnp.concatenate([qs_hi, qs_hi, qs_lo, ones, ones], axis=1)
    ccat = jnp.concatenate([c_hi, c_lo, c_hi, c2_hi, c2_lo], axis=1)
    g = jax.lax.dot_general(
        qcat, ccat, (((1,), (1,)), ((), ())),
        preferred_element_type=jnp.float32)
    m = jnp.min(g, axis=1)
    q2 = jnp.sum(q * q, axis=1)
    d2 = jnp.maximum(q2 + m, 0.0)
    o_ref[0, 0, pl.program_id(1)] = jnp.sum(jnp.sqrt(d2))


def kernel(pc1, pc2):
    b = pc1.shape[0]
    n = pc1.shape[1]
    qaos = jnp.stack([pc1[0], pc2[0], pc1[1], pc2[1]])
    caos = qaos[jnp.array([1, 0, 3, 2])]
    qsoa = jnp.stack([pc1[0].T, pc2[0].T, pc1[1].T, pc2[1].T])

    mesh = plsc.VectorSubcoreMesh(core_axis_name="c", subcore_axis_name="s")
    sc_run = functools.partial(
        pl.kernel,
        mesh=mesh,
        out_type=jax.ShapeDtypeStruct((32 * 16,), jnp.float32),
        scratch_types=[pltpu.VMEM((_CHUNK,), jnp.float32)] * 3
        + [pltpu.VMEM((_N,), jnp.float32)] * 3
        + [pltpu.VMEM((16,), jnp.float32)],
    )(_chamfer_sc)
    sc_part = sc_run(qsoa.reshape(-1))

    tc_q = qaos[:, _SC_Q:, :]
    ntiles = (n - _SC_Q) // _TC_TILE
    tc_part = pl.pallas_call(
        _chamfer_tc,
        grid=(_NCOMBO, ntiles),
        in_specs=[
            pl.BlockSpec((1, _TC_TILE, 3), lambda i, j: (i, j, 0)),
            pl.BlockSpec((1, n, 3), lambda i, j: (i, 0, 0)),
        ],
        out_specs=pl.BlockSpec(
            (1, 1, ntiles), lambda i, j: (i, 0, 0), memory_space=pltpu.SMEM),
        out_shape=jax.ShapeDtypeStruct((_NCOMBO, 1, ntiles), jnp.float32),
    )(tc_q, caos)

    total = jnp.sum(sc_part) + jnp.sum(tc_part)
    return total / jnp.float32(2 * n * b)


# speedup vs baseline: 1.8037x; 1.8037x over previous
import functools

import jax
import jax.numpy as jnp
from jax import lax
from jax.experimental import pallas as pl
from jax.experimental.pallas import tpu as pltpu
from jax.experimental.pallas import tpu_sc as plsc

_N = 4096
_NCOMBO = 4
_SC_Q = 1024
_CHUNK = _SC_Q // 8
_GBLK = 8
_TC_TILE = 256


def _newton_sqrt(x):
    i = lax.bitcast_convert_type(x, jnp.int32)
    y = lax.bitcast_convert_type(
        (i >> 1) + jnp.int32(0x1FBD3F7D), jnp.float32)
    half = jnp.float32(0.5)
    y = half * (y + x / y)
    y = half * (y + x / y)
    y = half * (y + x / y)
    return y


def _chamfer_sc(q_hbm, out_hbm, qx, qy, qz, cx, cy, cz, accv):
    nc = 2
    wid = lax.axis_index("s") * nc + lax.axis_index("c")
    combo = wid // 8
    chunk = wid % 8
    ccombo = combo ^ 1
    qbase = chunk * _CHUNK

    qrow = combo * (3 * _N)
    crow = ccombo * (3 * _N)
    pltpu.sync_copy(q_hbm.at[pl.ds(qrow + qbase, _CHUNK)], qx)
    pltpu.sync_copy(q_hbm.at[pl.ds(qrow + _N + qbase, _CHUNK)], qy)
    pltpu.sync_copy(q_hbm.at[pl.ds(qrow + 2 * _N + qbase, _CHUNK)], qz)
    pltpu.sync_copy(q_hbm.at[pl.ds(crow, _N)], cx)
    pltpu.sync_copy(q_hbm.at[pl.ds(crow + _N, _N)], cy)
    pltpu.sync_copy(q_hbm.at[pl.ds(crow + 2 * _N, _N)], cz)

    acc = jnp.zeros((16,), jnp.float32)
    big = jnp.full((16,), 3.0e38, jnp.float32)

    for blk in range(_CHUNK // 16 // _GBLK):
        qvs = []
        for g in range(_GBLK):
            off = (blk * _GBLK + g) * 16
            qvs.append((qx[pl.ds(off, 16)],
                        qy[pl.ds(off, 16)],
                        qz[pl.ds(off, 16)]))

        def body(jb, dmins, qvs=qvs):
            base = jb * 16
            cxv = cx[pl.ds(base, 16)]
            cyv = cy[pl.ds(base, 16)]
            czv = cz[pl.ds(base, 16)]
            out = list(dmins)
            for lane in range(16):
                bx = cxv[lane]
                by = cyv[lane]
                bz = czv[lane]
                for g in range(_GBLK):
                    dx = qvs[g][0] - bx
                    dy = qvs[g][1] - by
                    dz = qvs[g][2] - bz
                    d2 = dx * dx + dy * dy + dz * dz
                    out[g] = jnp.minimum(out[g], d2)
            return tuple(out)

        dmins = lax.fori_loop(0, _N // 16, body, tuple([big] * _GBLK))
        for g in range(_GBLK):
            acc = acc + _newton_sqrt(dmins[g])

    accv[...] = acc
    pltpu.sync_copy(accv, out_hbm.at[pl.ds(wid * 16, 16)])


def _chamfer_tc(q_ref, c_ref, o_ref):
    q = q_ref[0]
    c = c_ref[0]
    qs = q * (-2.0)
    qs_hi = qs.astype(jnp.bfloat16)
    qs_lo = (qs - qs_hi.astype(jnp.float32)).astype(jnp.bfloat16)
    c_hi = c.astype(jnp.bfloat16)
    c_lo = (c - c_hi.astype(jnp.float32)).astype(jnp.bfloat16)
    qcat = jnp.concatenate([qs_hi, qs_hi, qs_lo], axis=1)
    ccat = jnp.concatenate([c_hi, c_lo, c_hi], axis=0)
    g = jax.lax.dot_general(
        qcat, ccat, (((1,), (0,)), ((), ())),
        preferred_element_type=jnp.float32)
    c2 = jnp.sum(c * c, axis=0, keepdims=True)
    m = jnp.min(c2 + g, axis=1)
    q2 = jnp.sum(q * q, axis=1)
    d2 = jnp.maximum(q2 + m, 0.0)
    o_ref[0, 0, pl.program_id(1)] = jnp.sum(jnp.sqrt(d2))


def kernel(pc1, pc2):
    b = pc1.shape[0]
    n = pc1.shape[1]
    qaos = jnp.stack([pc1[0], pc2[0], pc1[1], pc2[1]])
    qsoa = jnp.stack([pc1[0].T, pc2[0].T, pc1[1].T, pc2[1].T])
    csoa = qsoa[jnp.array([1, 0, 3, 2])]

    mesh = plsc.VectorSubcoreMesh(core_axis_name="c", subcore_axis_name="s")
    sc_run = functools.partial(
        pl.kernel,
        mesh=mesh,
        out_type=jax.ShapeDtypeStruct((32 * 16,), jnp.float32),
        scratch_types=[pltpu.VMEM((_CHUNK,), jnp.float32)] * 3
        + [pltpu.VMEM((_N,), jnp.float32)] * 3
        + [pltpu.VMEM((16,), jnp.float32)],
    )(_chamfer_sc)
    sc_part = sc_run(qsoa.reshape(-1))

    tc_q = qaos[:, _SC_Q:, :]
    ntiles = (n - _SC_Q) // _TC_TILE
    tc_part = pl.pallas_call(
        _chamfer_tc,
        grid=(_NCOMBO, ntiles),
        in_specs=[
            pl.BlockSpec((1, _TC_TILE, 3), lambda i, j: (i, j, 0)),
            pl.BlockSpec((1, 3, n), lambda i, j: (i, 0, 0)),
        ],
        out_specs=pl.BlockSpec(
            (1, 1, ntiles), lambda i, j: (i, 0, 0), memory_space=pltpu.SMEM),
        out_shape=jax.ShapeDtypeStruct((_NCOMBO, 1, ntiles), jnp.float32),
    )(tc_q, csoa)

    total = jnp.sum(sc_part) + jnp.sum(tc_part)
    return total / jnp.float32(2 * n * b)


# speedup vs baseline: 1.8172x; 1.0075x over previous
import functools

import jax
import jax.numpy as jnp
from jax import lax
from jax.experimental import pallas as pl
from jax.experimental.pallas import tpu as pltpu
from jax.experimental.pallas import tpu_sc as plsc

_N = 4096
_NCOMBO = 4
_SC_Q = 1024
_CHUNK = _SC_Q // 8
_GBLK = 8
_TC_TILE = 256


def _newton_sqrt(x):
    i = lax.bitcast_convert_type(x, jnp.int32)
    y = lax.bitcast_convert_type(
        (i >> 1) + jnp.int32(0x1FBD3F7D), jnp.float32)
    half = jnp.float32(0.5)
    y = half * (y + x / y)
    y = half * (y + x / y)
    y = half * (y + x / y)
    return y


def _chamfer_sc(q_hbm, out_hbm, qx, qy, qz, cx, cy, cz, accv):
    nc = 2
    wid = lax.axis_index("s") * nc + lax.axis_index("c")
    combo = wid // 8
    chunk = wid % 8
    ccombo = combo ^ 1
    qbase = chunk * _CHUNK

    qrow = combo * (3 * _N)
    crow = ccombo * (3 * _N)
    pltpu.sync_copy(q_hbm.at[pl.ds(qrow + qbase, _CHUNK)], qx)
    pltpu.sync_copy(q_hbm.at[pl.ds(qrow + _N + qbase, _CHUNK)], qy)
    pltpu.sync_copy(q_hbm.at[pl.ds(qrow + 2 * _N + qbase, _CHUNK)], qz)
    pltpu.sync_copy(q_hbm.at[pl.ds(crow, _N)], cx)
    pltpu.sync_copy(q_hbm.at[pl.ds(crow + _N, _N)], cy)
    pltpu.sync_copy(q_hbm.at[pl.ds(crow + 2 * _N, _N)], cz)

    acc = jnp.zeros((16,), jnp.float32)
    big = jnp.full((16,), 3.0e38, jnp.float32)

    for blk in range(_CHUNK // 16 // _GBLK):
        qvs = []
        for g in range(_GBLK):
            off = (blk * _GBLK + g) * 16
            qvs.append((qx[pl.ds(off, 16)],
                        qy[pl.ds(off, 16)],
                        qz[pl.ds(off, 16)]))

        def body(jb, dmins, qvs=qvs):
            base = jb * 16
            cxv = cx[pl.ds(base, 16)]
            cyv = cy[pl.ds(base, 16)]
            czv = cz[pl.ds(base, 16)]
            out = list(dmins)
            for lane in range(16):
                bx = cxv[lane]
                by = cyv[lane]
                bz = czv[lane]
                for g in range(_GBLK):
                    dx = qvs[g][0] - bx
                    dy = qvs[g][1] - by
                    dz = qvs[g][2] - bz
                    d2 = dx * dx + dy * dy + dz * dz
                    out[g] = jnp.minimum(out[g], d2)
            return tuple(out)

        dmins = lax.fori_loop(0, _N // 16, body, tuple([big] * _GBLK),
                              unroll=2)
        for g in range(_GBLK):
            acc = acc + _newton_sqrt(dmins[g])

    accv[...] = acc
    pltpu.sync_copy(accv, out_hbm.at[pl.ds(wid * 16, 16)])


def _chamfer_tc(q_ref, c_ref, o_ref):
    q = q_ref[0]
    c = c_ref[0]
    qs = q * (-2.0)
    qs_hi = qs.astype(jnp.bfloat16)
    qs_lo = (qs - qs_hi.astype(jnp.float32)).astype(jnp.bfloat16)
    c_hi = c.astype(jnp.bfloat16)
    c_lo = (c - c_hi.astype(jnp.float32)).astype(jnp.bfloat16)
    qcat = jnp.concatenate([qs_hi, qs_hi, qs_lo], axis=1)
    ccat = jnp.concatenate([c_hi, c_lo, c_hi], axis=0)
    g = jax.lax.dot_general(
        qcat, ccat, (((1,), (0,)), ((), ())),
        preferred_element_type=jnp.float32)
    c2 = jnp.sum(c * c, axis=0, keepdims=True)
    m = jnp.min(c2 + g, axis=1)
    q2 = jnp.sum(q * q, axis=1)
    d2 = jnp.maximum(q2 + m, 0.0)
    o_ref[0, 0, pl.program_id(1)] = jnp.sum(jnp.sqrt(d2))


def kernel(pc1, pc2):
    b = pc1.shape[0]
    n = pc1.shape[1]
    qaos = jnp.stack([pc1[0], pc2[0], pc1[1], pc2[1]])
    qsoa = jnp.stack([pc1[0].T, pc2[0].T, pc1[1].T, pc2[1].T])
    csoa = qsoa[jnp.array([1, 0, 3, 2])]

    mesh = plsc.VectorSubcoreMesh(core_axis_name="c", subcore_axis_name="s")
    sc_run = functools.partial(
        pl.kernel,
        mesh=mesh,
        out_type=jax.ShapeDtypeStruct((32 * 16,), jnp.float32),
        scratch_types=[pltpu.VMEM((_CHUNK,), jnp.float32)] * 3
        + [pltpu.VMEM((_N,), jnp.float32)] * 3
        + [pltpu.VMEM((16,), jnp.float32)],
    )(_chamfer_sc)
    sc_part = sc_run(qsoa.reshape(-1))

    tc_q = qaos[:, _SC_Q:, :]
    ntiles = (n - _SC_Q) // _TC_TILE
    tc_part = pl.pallas_call(
        _chamfer_tc,
        grid=(_NCOMBO, ntiles),
        in_specs=[
            pl.BlockSpec((1, _TC_TILE, 3), lambda i, j: (i, j, 0)),
            pl.BlockSpec((1, 3, n), lambda i, j: (i, 0, 0)),
        ],
        out_specs=pl.BlockSpec(
            (1, 1, ntiles), lambda i, j: (i, 0, 0), memory_space=pltpu.SMEM),
        out_shape=jax.ShapeDtypeStruct((_NCOMBO, 1, ntiles), jnp.float32),
    )(tc_q, csoa)

    total = jnp.sum(sc_part) + jnp.sum(tc_part)
    return total / jnp.float32(2 * n * b)


# speedup vs baseline: 1.8216x; 1.0024x over previous
import functools

import jax
import jax.numpy as jnp
from jax import lax
from jax.experimental import pallas as pl
from jax.experimental.pallas import tpu as pltpu
from jax.experimental.pallas import tpu_sc as plsc

_N = 4096
_NCOMBO = 4
_SC_Q = 1024
_CHUNK = _SC_Q // 8
_GBLK = 8
_TC_TILE = 512


def _newton_sqrt(x):
    i = lax.bitcast_convert_type(x, jnp.int32)
    y = lax.bitcast_convert_type(
        (i >> 1) + jnp.int32(0x1FBD3F7D), jnp.float32)
    half = jnp.float32(0.5)
    y = half * (y + x / y)
    y = half * (y + x / y)
    y = half * (y + x / y)
    return y


def _chamfer_sc(q_hbm, out_hbm, qx, qy, qz, cx, cy, cz, accv):
    nc = 2
    wid = lax.axis_index("s") * nc + lax.axis_index("c")
    combo = wid // 8
    chunk = wid % 8
    ccombo = combo ^ 1
    qbase = chunk * _CHUNK

    qrow = combo * (3 * _N)
    crow = ccombo * (3 * _N)
    pltpu.sync_copy(q_hbm.at[pl.ds(qrow + qbase, _CHUNK)], qx)
    pltpu.sync_copy(q_hbm.at[pl.ds(qrow + _N + qbase, _CHUNK)], qy)
    pltpu.sync_copy(q_hbm.at[pl.ds(qrow + 2 * _N + qbase, _CHUNK)], qz)
    pltpu.sync_copy(q_hbm.at[pl.ds(crow, _N)], cx)
    pltpu.sync_copy(q_hbm.at[pl.ds(crow + _N, _N)], cy)
    pltpu.sync_copy(q_hbm.at[pl.ds(crow + 2 * _N, _N)], cz)

    acc = jnp.zeros((16,), jnp.float32)
    big = jnp.full((16,), 3.0e38, jnp.float32)

    for blk in range(_CHUNK // 16 // _GBLK):
        qvs = []
        for g in range(_GBLK):
            off = (blk * _GBLK + g) * 16
            qvs.append((qx[pl.ds(off, 16)],
                        qy[pl.ds(off, 16)],
                        qz[pl.ds(off, 16)]))

        def body(jb, dmins, qvs=qvs):
            base = jb * 16
            cxv = cx[pl.ds(base, 16)]
            cyv = cy[pl.ds(base, 16)]
            czv = cz[pl.ds(base, 16)]
            out = list(dmins)
            for lane in range(16):
                bx = cxv[lane]
                by = cyv[lane]
                bz = czv[lane]
                for g in range(_GBLK):
                    dx = qvs[g][0] - bx
                    dy = qvs[g][1] - by
                    dz = qvs[g][2] - bz
                    d2 = dx * dx + dy * dy + dz * dz
                    out[g] = jnp.minimum(out[g], d2)
            return tuple(out)

        dmins = lax.fori_loop(0, _N // 16, body, tuple([big] * _GBLK),
                              unroll=2)
        for g in range(_GBLK):
            acc = acc + _newton_sqrt(dmins[g])

    accv[...] = acc
    pltpu.sync_copy(accv, out_hbm.at[pl.ds(wid * 16, 16)])


def _chamfer_tc(q_ref, c_ref, o_ref):
    q = q_ref[0]
    c = c_ref[0]
    qs = q * (-2.0)
    qs_hi = qs.astype(jnp.bfloat16)
    qs_lo = (qs - qs_hi.astype(jnp.float32)).astype(jnp.bfloat16)
    c_hi = c.astype(jnp.bfloat16)
    c_lo = (c - c_hi.astype(jnp.float32)).astype(jnp.bfloat16)
    qcat = jnp.concatenate([qs_hi, qs_hi, qs_lo], axis=1)
    ccat = jnp.concatenate([c_hi, c_lo, c_hi], axis=0)
    g = jax.lax.dot_general(
        qcat, ccat, (((1,), (0,)), ((), ())),
        preferred_element_type=jnp.float32)
    c2 = jnp.sum(c * c, axis=0, keepdims=True)
    m = jnp.min(c2 + g, axis=1)
    q2 = jnp.sum(q * q, axis=1)
    d2 = jnp.maximum(q2 + m, 0.0)
    o_ref[0, 0, pl.program_id(1)] = jnp.sum(jnp.sqrt(d2))


def kernel(pc1, pc2):
    b = pc1.shape[0]
    n = pc1.shape[1]
    qaos = jnp.stack([pc1[0], pc2[0], pc1[1], pc2[1]])
    qsoa = jnp.stack([pc1[0].T, pc2[0].T, pc1[1].T, pc2[1].T])
    csoa = qsoa[jnp.array([1, 0, 3, 2])]

    mesh = plsc.VectorSubcoreMesh(core_axis_name="c", subcore_axis_name="s")
    sc_run = functools.partial(
        pl.kernel,
        mesh=mesh,
        out_type=jax.ShapeDtypeStruct((32 * 16,), jnp.float32),
        scratch_types=[pltpu.VMEM((_CHUNK,), jnp.float32)] * 3
        + [pltpu.VMEM((_N,), jnp.float32)] * 3
        + [pltpu.VMEM((16,), jnp.float32)],
    )(_chamfer_sc)
    sc_part = sc_run(qsoa.reshape(-1))

    tc_q = qaos[:, _SC_Q:, :]
    ntiles = (n - _SC_Q) // _TC_TILE
    tc_part = pl.pallas_call(
        _chamfer_tc,
        grid=(_NCOMBO, ntiles),
        in_specs=[
            pl.BlockSpec((1, _TC_TILE, 3), lambda i, j: (i, j, 0)),
            pl.BlockSpec((1, 3, n), lambda i, j: (i, 0, 0)),
        ],
        out_specs=pl.BlockSpec(
            (1, 1, ntiles), lambda i, j: (i, 0, 0), memory_space=pltpu.SMEM),
        out_shape=jax.ShapeDtypeStruct((_NCOMBO, 1, ntiles), jnp.float32),
    )(tc_q, csoa)

    total = jnp.sum(sc_part) + jnp.sum(tc_part)
    return total / jnp.float32(2 * n * b)
